# baseline (speedup 1.0000x reference)
import jax
import jax.numpy as jnp
from jax.experimental import pallas as pl
from jax.experimental.pallas import tpu as pltpu

_DIM = 128
_K = 65536
_BATCH = 4096
_NBLK = _K // _BATCH


def _body(ptr_ref, keys_ref, queue_ref, out_ref, kt, csem, ksem, osem):
    ptr = pl.multiple_of(ptr_ref[0], _BATCH)
    copies = []
    for i in range(_NBLK):
        @pl.when(i * _BATCH != ptr)
        def _():
            h = pltpu.make_async_copy(
                queue_ref.at[:, pl.ds(i * _BATCH, _BATCH)],
                out_ref.at[:, pl.ds(i * _BATCH, _BATCH)],
                csem,
            )
            h.start()

        copies.append(
            pltpu.make_async_copy(
                queue_ref.at[:, pl.ds(0, _BATCH)],
                out_ref.at[:, pl.ds(0, _BATCH)],
                csem,
            )
        )
    kin = pltpu.make_async_copy(keys_ref, kt, ksem)
    kin.start()
    kin.wait()

    def scoped(tref):
        tref[...] = kt[...].T
        kout = pltpu.make_async_copy(tref, out_ref.at[:, pl.ds(ptr, _BATCH)], osem)
        kout.start()
        kout.wait()

    pl.run_scoped(scoped, pltpu.VMEM((_DIM, _BATCH), jnp.float32))
    for _ in range(_NBLK - 1):
        copies[0].wait()


def kernel(keys, queue, queue_ptr):
    ptr = jnp.asarray(queue_ptr, jnp.int32).reshape((1,))
    new_queue = pl.pallas_call(
        _body,
        in_specs=[
            pl.BlockSpec(memory_space=pltpu.SMEM),
            pl.BlockSpec(memory_space=pl.ANY),
            pl.BlockSpec(memory_space=pl.ANY),
        ],
        out_specs=pl.BlockSpec(memory_space=pl.ANY),
        out_shape=jax.ShapeDtypeStruct((_DIM, _K), jnp.float32),
        scratch_shapes=[
            pltpu.VMEM((_BATCH, _DIM), jnp.float32),
            pltpu.SemaphoreType.DMA,
            pltpu.SemaphoreType.DMA,
            pltpu.SemaphoreType.DMA,
        ],
    )(ptr, keys, queue)
    new_ptr = (jnp.asarray(queue_ptr, jnp.int32) + _BATCH) % _K
    return new_queue, jnp.asarray(new_ptr, dtype=jnp.int64)


# speedup vs baseline: 26.5137x; 26.5137x over previous
import jax
import jax.numpy as jnp
from jax.experimental import pallas as pl
from jax.experimental.pallas import tpu as pltpu

_DIM = 128
_K = 65536
_BATCH = 4096
_CW = 2048
_NCH = _K // _CW
_NBUF = 6


def _body(ptr_ref, keys_ref, queue_ref, out_ref, bufs, kt, isems, osems, ksem):
    ptr = pl.multiple_of(ptr_ref[0], _BATCH)

    def in_copy(c, b):
        return pltpu.make_async_copy(
            queue_ref.at[:, pl.ds(c * _CW, _CW)], bufs.at[b], isems.at[b]
        )

    def out_copy(c, b):
        return pltpu.make_async_copy(
            bufs.at[b], out_ref.at[:, pl.ds(c * _CW, _CW)], osems.at[b]
        )

    kin = pltpu.make_async_copy(keys_ref, kt, ksem)
    kin.start()
    for b in range(_NBUF):
        in_copy(b, b).start()
    for c in range(_NCH):
        b = c % _NBUF
        in_copy(c, b).wait()
        out_copy(c, b).start()
        if c + _NBUF < _NCH:
            out_copy(c, b).wait()
            in_copy(c + _NBUF, b).start()
    for c in range(_NCH - _NBUF, _NCH):
        out_copy(c, c % _NBUF).wait()
    kin.wait()

    def scoped(tref):
        tref[...] = kt[...].T
        kout = pltpu.make_async_copy(tref, out_ref.at[:, pl.ds(ptr, _BATCH)], ksem)
        kout.start()
        kout.wait()

    pl.run_scoped(scoped, pltpu.VMEM((_DIM, _BATCH), jnp.float32))


def kernel(keys, queue, queue_ptr):
    ptr = jnp.asarray(queue_ptr, jnp.int32).reshape((1,))
    new_queue = pl.pallas_call(
        _body,
        in_specs=[
            pl.BlockSpec(memory_space=pltpu.SMEM),
            pl.BlockSpec(memory_space=pl.ANY),
            pl.BlockSpec(memory_space=pl.ANY),
        ],
        out_specs=pl.BlockSpec(memory_space=pl.ANY),
        out_shape=jax.ShapeDtypeStruct((_DIM, _K), jnp.float32),
        scratch_shapes=[
            pltpu.VMEM((_NBUF, _DIM, _CW), jnp.float32),
            pltpu.VMEM((_BATCH, _DIM), jnp.float32),
            pltpu.SemaphoreType.DMA((_NBUF,)),
            pltpu.SemaphoreType.DMA((_NBUF,)),
            pltpu.SemaphoreType.DMA,
        ],
    )(ptr, keys, queue)
    new_ptr = (jnp.asarray(queue_ptr, jnp.int32) + _BATCH) % _K
    return new_queue, jnp.asarray(new_ptr, dtype=jnp.int64)


# speedup vs baseline: 34.0270x; 1.2834x over previous
import jax
import jax.numpy as jnp
from jax.experimental import pallas as pl
from jax.experimental.pallas import tpu as pltpu

_DIM = 128
_K = 65536
_BATCH = 4096
_RH = 8
_NCH = _DIM // _RH
_NBUF = 4


def _body(ptr_ref, keys_ref, queue_ref, out_ref, bufs, kt, isems, osems, ksem):
    ptr = pl.multiple_of(ptr_ref[0], _BATCH)

    def in_copy(c, b):
        return pltpu.make_async_copy(
            queue_ref.at[pl.ds(c * _RH, _RH), :], bufs.at[b], isems.at[b]
        )

    def out_copy(c, b):
        return pltpu.make_async_copy(
            bufs.at[b], out_ref.at[pl.ds(c * _RH, _RH), :], osems.at[b]
        )

    kin = pltpu.make_async_copy(keys_ref, kt, ksem)
    kin.start()
    for b in range(_NBUF):
        in_copy(b, b).start()
    for c in range(_NCH):
        b = c % _NBUF
        in_copy(c, b).wait()
        out_copy(c, b).start()
        if c + _NBUF < _NCH:
            out_copy(c, b).wait()
            in_copy(c + _NBUF, b).start()
    for c in range(_NCH - _NBUF, _NCH):
        out_copy(c, c % _NBUF).wait()
    kin.wait()

    def scoped(tref):
        tref[...] = kt[...].T
        kout = pltpu.make_async_copy(tref, out_ref.at[:, pl.ds(ptr, _BATCH)], ksem)
        kout.start()
        kout.wait()

    pl.run_scoped(scoped, pltpu.VMEM((_DIM, _BATCH), jnp.float32))


def kernel(keys, queue, queue_ptr):
    ptr = jnp.asarray(queue_ptr, jnp.int32).reshape((1,))
    new_queue = pl.pallas_call(
        _body,
        in_specs=[
            pl.BlockSpec(memory_space=pltpu.SMEM),
            pl.BlockSpec(memory_space=pl.ANY),
            pl.BlockSpec(memory_space=pl.ANY),
        ],
        out_specs=pl.BlockSpec(memory_space=pl.ANY),
        out_shape=jax.ShapeDtypeStruct((_DIM, _K), jnp.float32),
        scratch_shapes=[
            pltpu.VMEM((_NBUF, _RH, _K), jnp.float32),
            pltpu.VMEM((_BATCH, _DIM), jnp.float32),
            pltpu.SemaphoreType.DMA((_NBUF,)),
            pltpu.SemaphoreType.DMA((_NBUF,)),
            pltpu.SemaphoreType.DMA,
        ],
    )(ptr, keys, queue)
    new_ptr = (jnp.asarray(queue_ptr, jnp.int32) + _BATCH) % _K
    return new_queue, jnp.asarray(new_ptr, dtype=jnp.int64)
